# baseline (speedup 1.0000x reference)
import jax
import jax.numpy as jnp
from jax import lax
from jax.experimental import pallas as pl
from jax.experimental.pallas import tpu as pltpu

N_BS = 200


def _deg_kernel(adj_ref, out_ref):
    s = pl.program_id(0)

    @pl.when(s == 0)
    def _():
        out_ref[...] = jnp.zeros_like(out_ref)

    out_ref[...] += jnp.sum(adj_ref[...], axis=0, keepdims=True)


def _y1_kernel(x_ref, w1_ref, deg_ref, y1_ref):
    xw = jnp.dot(x_ref[...], w1_ref[...], preferred_element_type=jnp.float32)
    dinv = lax.rsqrt(deg_ref[...])
    y1_ref[...] = dinv.T * xw


def _layer1_kernel(adj_ref, y1_ref, y1full_ref, deg_ref, b1_ref, w2_ref,
                   y2_ref, acc_ref):
    s = pl.program_id(0)

    @pl.when(s == 0)
    def _():
        acc_ref[...] = jnp.zeros_like(acc_ref)

    acc_ref[...] += lax.dot_general(
        adj_ref[...], y1_ref[...], (((0,), (0,)), ((), ())),
        preferred_element_type=jnp.float32)

    @pl.when(s == pl.num_programs(0) - 1)
    def _():
        dinv = lax.rsqrt(deg_ref[...]).T
        pre = dinv * (acc_ref[...] + y1full_ref[...]) + b1_ref[...]
        h = jnp.where(pre > 0, pre, jnp.exp(pre) - 1.0)
        xw2 = jnp.dot(h, w2_ref[...], preferred_element_type=jnp.float32)
        y2_ref[...] = dinv * xw2


def _layer2_kernel(adj_ref, y2_ref, y2full_ref, deg_ref, b2_ref,
                   out_ref, acc_ref):
    s = pl.program_id(0)

    @pl.when(s == 0)
    def _():
        acc_ref[...] = jnp.zeros_like(acc_ref)

    acc_ref[...] += lax.dot_general(
        adj_ref[...], y2_ref[...], (((0,), (0,)), ((), ())),
        preferred_element_type=jnp.float32)

    @pl.when(s == pl.num_programs(0) - 1)
    def _():
        dinv = lax.rsqrt(deg_ref[...]).T
        out_ref[...] = dinv * (acc_ref[...] + y2full_ref[...]) + b2_ref[...]


def _gcn_two_layer(x, adj, W1, b1, W2, b2, interpret=False):
    n, d_in = x.shape
    d_hid = W1.shape[1]
    d_out = W2.shape[1]
    ns = n // N_BS
    nd = pl.cdiv(n, 512)

    colsum = pl.pallas_call(
        _deg_kernel,
        grid=(ns,),
        in_specs=[pl.BlockSpec((N_BS, n), lambda s: (s, 0))],
        out_specs=pl.BlockSpec((1, n), lambda s: (0, 0)),
        out_shape=jax.ShapeDtypeStruct((1, n), jnp.float32),
        interpret=interpret,
    )(adj)
    deg = colsum + 1.0

    y1 = pl.pallas_call(
        _y1_kernel,
        grid=(nd,),
        in_specs=[
            pl.BlockSpec((512, d_in), lambda d: (d, 0)),
            pl.BlockSpec((d_in, d_hid), lambda d: (0, 0)),
            pl.BlockSpec((1, 512), lambda d: (0, d)),
        ],
        out_specs=pl.BlockSpec((512, d_hid), lambda d: (d, 0)),
        out_shape=jax.ShapeDtypeStruct((n, d_hid), jnp.float32),
        interpret=interpret,
    )(x, W1, deg)

    y2 = pl.pallas_call(
        _layer1_kernel,
        grid=(ns,),
        in_specs=[
            pl.BlockSpec((N_BS, n), lambda s: (s, 0)),
            pl.BlockSpec((N_BS, d_hid), lambda s: (s, 0)),
            pl.BlockSpec((n, d_hid), lambda s: (0, 0)),
            pl.BlockSpec((1, n), lambda s: (0, 0)),
            pl.BlockSpec((1, d_hid), lambda s: (0, 0)),
            pl.BlockSpec((d_hid, d_out), lambda s: (0, 0)),
        ],
        out_specs=pl.BlockSpec((n, d_out), lambda s: (0, 0)),
        out_shape=jax.ShapeDtypeStruct((n, d_out), jnp.float32),
        scratch_shapes=[pltpu.VMEM((n, d_hid), jnp.float32)],
        interpret=interpret,
    )(adj, y1, y1, deg, b1.reshape(1, -1), W2)

    out = pl.pallas_call(
        _layer2_kernel,
        grid=(ns,),
        in_specs=[
            pl.BlockSpec((N_BS, n), lambda s: (s, 0)),
            pl.BlockSpec((N_BS, d_out), lambda s: (s, 0)),
            pl.BlockSpec((n, d_out), lambda s: (0, 0)),
            pl.BlockSpec((1, n), lambda s: (0, 0)),
            pl.BlockSpec((1, d_out), lambda s: (0, 0)),
        ],
        out_specs=pl.BlockSpec((n, d_out), lambda s: (0, 0)),
        out_shape=jax.ShapeDtypeStruct((n, d_out), jnp.float32),
        scratch_shapes=[pltpu.VMEM((n, d_out), jnp.float32)],
        interpret=interpret,
    )(adj, y2, y2, deg, b2.reshape(1, -1))

    return out


def kernel(x, adj_matrix, W1, b1, W2, b2):
    outs = [
        _gcn_two_layer(x[bi], adj_matrix[bi], W1, b1, W2, b2)
        for bi in range(x.shape[0])
    ]
    return jnp.stack(outs, axis=0)


# speedup vs baseline: 1.0772x; 1.0772x over previous
import jax
import jax.numpy as jnp
from jax import lax
from jax.experimental import pallas as pl
from jax.experimental.pallas import tpu as pltpu

N_BS = 400


def _deg_kernel(adj_ref, out_ref):
    s = pl.program_id(0)

    @pl.when(s == 0)
    def _():
        out_ref[...] = jnp.zeros_like(out_ref)

    out_ref[...] += jnp.sum(adj_ref[...], axis=0, keepdims=True)


def _y1_kernel(x_ref, w1_ref, deg_ref, y1_ref):
    xw = jnp.dot(x_ref[...], w1_ref[...], preferred_element_type=jnp.float32)
    dinv = lax.rsqrt(deg_ref[...])
    y1_ref[...] = dinv.T * xw


def _layer1_kernel(adj_ref, y1_ref, y1full_ref, deg_ref, b1_ref, w2_ref,
                   y2_ref, acc_ref):
    s = pl.program_id(0)

    @pl.when(s == 0)
    def _():
        acc_ref[...] = jnp.zeros_like(acc_ref)

    acc_ref[...] += lax.dot_general(
        adj_ref[...], y1_ref[...], (((0,), (0,)), ((), ())),
        preferred_element_type=jnp.float32)

    @pl.when(s == pl.num_programs(0) - 1)
    def _():
        dinv = lax.rsqrt(deg_ref[...]).T
        pre = dinv * (acc_ref[...] + y1full_ref[...]) + b1_ref[...]
        h = jnp.where(pre > 0, pre, jnp.exp(pre) - 1.0)
        xw2 = jnp.dot(h, w2_ref[...], preferred_element_type=jnp.float32)
        y2_ref[...] = dinv * xw2


def _layer2_kernel(adj_ref, y2_ref, y2full_ref, deg_ref, b2_ref,
                   out_ref, acc_ref):
    s = pl.program_id(0)

    @pl.when(s == 0)
    def _():
        acc_ref[...] = jnp.zeros_like(acc_ref)

    acc_ref[...] += lax.dot_general(
        adj_ref[...], y2_ref[...], (((0,), (0,)), ((), ())),
        preferred_element_type=jnp.float32)

    @pl.when(s == pl.num_programs(0) - 1)
    def _():
        dinv = lax.rsqrt(deg_ref[...]).T
        out_ref[...] = dinv * (acc_ref[...] + y2full_ref[...]) + b2_ref[...]


def _gcn_two_layer(x, adj, W1, b1, W2, b2, interpret=False):
    n, d_in = x.shape
    d_hid = W1.shape[1]
    d_out = W2.shape[1]
    ns = n // N_BS
    nd = pl.cdiv(n, 512)

    colsum = pl.pallas_call(
        _deg_kernel,
        grid=(ns,),
        in_specs=[pl.BlockSpec((N_BS, n), lambda s: (s, 0))],
        out_specs=pl.BlockSpec((1, n), lambda s: (0, 0)),
        out_shape=jax.ShapeDtypeStruct((1, n), jnp.float32),
        interpret=interpret,
    )(adj)
    deg = colsum + 1.0

    y1 = pl.pallas_call(
        _y1_kernel,
        grid=(nd,),
        in_specs=[
            pl.BlockSpec((512, d_in), lambda d: (d, 0)),
            pl.BlockSpec((d_in, d_hid), lambda d: (0, 0)),
            pl.BlockSpec((1, 512), lambda d: (0, d)),
        ],
        out_specs=pl.BlockSpec((512, d_hid), lambda d: (d, 0)),
        out_shape=jax.ShapeDtypeStruct((n, d_hid), jnp.float32),
        interpret=interpret,
    )(x, W1, deg)

    y2 = pl.pallas_call(
        _layer1_kernel,
        grid=(ns,),
        in_specs=[
            pl.BlockSpec((N_BS, n), lambda s: (s, 0)),
            pl.BlockSpec((N_BS, d_hid), lambda s: (s, 0)),
            pl.BlockSpec((n, d_hid), lambda s: (0, 0)),
            pl.BlockSpec((1, n), lambda s: (0, 0)),
            pl.BlockSpec((1, d_hid), lambda s: (0, 0)),
            pl.BlockSpec((d_hid, d_out), lambda s: (0, 0)),
        ],
        out_specs=pl.BlockSpec((n, d_out), lambda s: (0, 0)),
        out_shape=jax.ShapeDtypeStruct((n, d_out), jnp.float32),
        scratch_shapes=[pltpu.VMEM((n, d_hid), jnp.float32)],
        interpret=interpret,
    )(adj, y1, y1, deg, b1.reshape(1, -1), W2)

    out = pl.pallas_call(
        _layer2_kernel,
        grid=(ns,),
        in_specs=[
            pl.BlockSpec((N_BS, n), lambda s: (s, 0)),
            pl.BlockSpec((N_BS, d_out), lambda s: (s, 0)),
            pl.BlockSpec((n, d_out), lambda s: (0, 0)),
            pl.BlockSpec((1, n), lambda s: (0, 0)),
            pl.BlockSpec((1, d_out), lambda s: (0, 0)),
        ],
        out_specs=pl.BlockSpec((n, d_out), lambda s: (0, 0)),
        out_shape=jax.ShapeDtypeStruct((n, d_out), jnp.float32),
        scratch_shapes=[pltpu.VMEM((n, d_out), jnp.float32)],
        interpret=interpret,
    )(adj, y2, y2, deg, b2.reshape(1, -1))

    return out


def kernel(x, adj_matrix, W1, b1, W2, b2):
    outs = [
        _gcn_two_layer(x[bi], adj_matrix[bi], W1, b1, W2, b2)
        for bi in range(x.shape[0])
    ]
    return jnp.stack(outs, axis=0)
